# baseline (speedup 1.0000x reference)
import functools

import jax
import jax.numpy as jnp
import numpy as np
from jax.experimental import pallas as pl


def _leaky(z):
    return jnp.where(z > 0, z, 0.01 * z)


def _mm_kernel(a_ref, w_ref, b_ref, o_ref, *, act):
    z = jnp.dot(a_ref[...], w_ref[...], preferred_element_type=jnp.float32)
    z = z + b_ref[...]
    if act:
        z = jnp.where(z > 0, z, 0.01 * z)
    o_ref[...] = z


def _mm(a, w, b, act=False, bm=1024):
    m, k = a.shape
    f = w.shape[1]
    mp = ((m + bm - 1) // bm) * bm
    if mp != m:
        a = jnp.pad(a, ((0, mp - m), (0, 0)))
    fp = max(128, ((f + 127) // 128) * 128)
    if fp != f:
        w = jnp.pad(w, ((0, 0), (0, fp - f)))
        b = jnp.pad(b, ((0, fp - f),))
    out = pl.pallas_call(
        functools.partial(_mm_kernel, act=act),
        grid=(mp // bm,),
        in_specs=[
            pl.BlockSpec((bm, k), lambda i: (i, 0)),
            pl.BlockSpec((k, fp), lambda i: (0, 0)),
            pl.BlockSpec((fp,), lambda i: (0,)),
        ],
        out_specs=pl.BlockSpec((bm, fp), lambda i: (i, 0)),
        out_shape=jax.ShapeDtypeStruct((mp, fp), jnp.float32),
    )(a, w, b)
    return out[:m, :f]


def kernel(x, edge_attr, W0, b0, Wq, bq, Wk, bk, Wv, bv, We, be, Ws, bs,
           Ws1, bs1, Ws2, bs2, Ws3, bs3, Wm1, bm1, Wm2, bm2, Wm3, bm3,
           edge_index, batch, stems, stems_batch, slices_x):
    n = x.shape[0]
    D = Wq.shape[1]
    G = slices_x.shape[0] - 1
    src = edge_index[0]
    dst = edge_index[1]
    scale = 1.0 / np.sqrt(D)

    out = _mm(x, W0, b0, act=True)
    e = edge_attr @ We + be
    for _ in range(12):
        q = _mm(out, Wq, bq)
        k = _mm(out, Wk, bk)
        v = _mm(out, Wv, bv)
        logits = jnp.sum(q[dst] * (k[src] + e), axis=-1) * scale
        m = jax.ops.segment_max(logits, dst, num_segments=n)
        a = jnp.exp(logits - m[dst])
        denom = jax.ops.segment_sum(a, dst, num_segments=n)
        alpha = a / (denom[dst] + 1e-16)
        msg = (v[src] + e) * alpha[:, None]
        agg = jax.ops.segment_sum(msg, dst, num_segments=n)
        out = _leaky(agg + _mm(out, Ws, bs))

    npg = n // G
    global_out = out.reshape(G, npg, D).mean(axis=1)
    stem_idx = slices_x[stems_batch] + stems
    stem_atom_out = out[stem_idx]
    stem_in = jnp.concatenate([stem_atom_out, global_out[stems_batch]], axis=1)
    h = _mm(stem_in, Ws1, bs1, act=True, bm=512)
    h = _mm(h, Ws2, bs2, act=True, bm=512)
    per_stem_out = _mm(h, Ws3, bs3, bm=512)
    g = _leaky(global_out @ Wm1 + bm1)
    g = _leaky(g @ Wm2 + bm2)
    per_mol_out = g @ Wm3 + bm3
    return per_stem_out, per_mol_out


# speedup vs baseline: 1.1634x; 1.1634x over previous
import functools

import jax
import jax.numpy as jnp
import numpy as np
from jax import lax
from jax.experimental import pallas as pl
from jax.experimental.pallas import tpu as pltpu
from jax.experimental.pallas import tpu_sc as plsc

_NW = 32
_NSUB = 16
_CH = 64
_LBUF = 8192
_D = 256
_AW = 16
_ROW = _D + _AW


def _wid():
    return lax.axis_index("s") * 2 + lax.axis_index("c")


def _iota16():
    return lax.iota(jnp.int32, 16)


def _sget(vec, j):
    return vec[j]


def _smax(vec):
    return plsc.cummax(vec)[15]


def _leaky(z):
    return jnp.where(z > 0, z, 0.01 * z)


def _mm_kernel(a_ref, w_ref, b_ref, o_ref, *, act):
    z = jnp.dot(a_ref[...], w_ref[...], preferred_element_type=jnp.float32)
    z = z + b_ref[...]
    if act:
        z = jnp.where(z > 0, z, 0.01 * z)
    o_ref[...] = z


def _mm(a, w, b, act=False, bm=1024):
    m, k = a.shape
    f = w.shape[1]
    fp = max(128, ((f + 127) // 128) * 128)
    if fp != f:
        w = jnp.pad(w, ((0, 0), (0, fp - f)))
        b = jnp.pad(b, ((0, fp - f),))
    grid = (m + bm - 1) // bm
    out = pl.pallas_call(
        functools.partial(_mm_kernel, act=act),
        grid=(grid,),
        in_specs=[
            pl.BlockSpec((bm, k), lambda i: (i, 0)),
            pl.BlockSpec((k, fp), lambda i: (0, 0)),
            pl.BlockSpec((fp,), lambda i: (0,)),
        ],
        out_specs=pl.BlockSpec((bm, fp), lambda i: (i, 0)),
        out_shape=jax.ShapeDtypeStruct((m, fp), jnp.float32),
    )(a, w, b)
    return out[:, :f] if fp != f else out


def _qkv_kernel(h_ref, w3_ref, b3_ref, we_ref, q_ref, k_ref, v_ref, qe_ref):
    z = jnp.dot(h_ref[...], w3_ref[...], preferred_element_type=jnp.float32)
    z = z + b3_ref[...]
    q = z[:, :_D]
    q_ref[...] = q
    k_ref[...] = z[:, _D:2 * _D]
    v_ref[...] = z[:, 2 * _D:]
    qe_ref[...] = jnp.dot(q, we_ref[...], preferred_element_type=jnp.float32)


def _qkv(h, w3, b3, we16, n_out=None, bm=512):
    n = n_out if n_out is not None else h.shape[0]
    grid = (n + bm - 1) // bm
    return pl.pallas_call(
        _qkv_kernel,
        grid=(grid,),
        in_specs=[
            pl.BlockSpec((bm, _D), lambda i: (i, 0)),
            pl.BlockSpec((_D, 3 * _D), lambda i: (0, 0)),
            pl.BlockSpec((3 * _D,), lambda i: (0,)),
            pl.BlockSpec((_D, _AW), lambda i: (0, 0)),
        ],
        out_specs=[
            pl.BlockSpec((bm, _D), lambda i: (i, 0)),
            pl.BlockSpec((bm, _D), lambda i: (i, 0)),
            pl.BlockSpec((bm, _D), lambda i: (i, 0)),
            pl.BlockSpec((bm, _AW), lambda i: (i, 0)),
        ],
        out_shape=[
            jax.ShapeDtypeStruct((n, _D), jnp.float32),
            jax.ShapeDtypeStruct((n, _D), jnp.float32),
            jax.ShapeDtypeStruct((n, _D), jnp.float32),
            jax.ShapeDtypeStruct((n, _AW), jnp.float32),
        ],
    )(h, w3, b3, we16)


def _update_kernel(aggv_ref, awa_ref, h_ref, ws_ref, wf_ref, bs_ref, o_ref):
    z = jnp.dot(h_ref[...], ws_ref[...], preferred_element_type=jnp.float32)
    z = z + jnp.dot(awa_ref[...], wf_ref[...], preferred_element_type=jnp.float32)
    z = z + aggv_ref[...] + bs_ref[...]
    o_ref[...] = jnp.where(z > 0, z, 0.01 * z)


def _update(aggv, awa, h, ws, wfold, bs, bm=512):
    n = h.shape[0]
    grid = (n + bm - 1) // bm
    return pl.pallas_call(
        _update_kernel,
        grid=(grid,),
        in_specs=[
            pl.BlockSpec((bm, _D), lambda i: (i, 0)),
            pl.BlockSpec((bm, _AW), lambda i: (i, 0)),
            pl.BlockSpec((bm, _D), lambda i: (i, 0)),
            pl.BlockSpec((_D, _D), lambda i: (0, 0)),
            pl.BlockSpec((_AW, _D), lambda i: (0, 0)),
            pl.BlockSpec((_D,), lambda i: (0,)),
        ],
        out_specs=pl.BlockSpec((bm, _D), lambda i: (i, 0)),
        out_shape=jax.ShapeDtypeStruct((n, _D), jnp.float32),
    )(aggv, awa, h, ws, wfold, bs)


def _pool_kernel(x_ref, o_ref, *, npg, gpb):
    rows = x_ref.shape[0]
    gi = lax.broadcasted_iota(jnp.int32, (gpb, rows), 0)
    ri = lax.broadcasted_iota(jnp.int32, (gpb, rows), 1)
    m = (ri // npg == gi).astype(jnp.float32)
    o_ref[...] = jnp.dot(m, x_ref[...], preferred_element_type=jnp.float32) / npg


def _pool(x, g, npg, gpb=8):
    n = x.shape[0]
    grid = (g + gpb - 1) // gpb
    out = pl.pallas_call(
        functools.partial(_pool_kernel, npg=npg, gpb=gpb),
        grid=(grid,),
        in_specs=[pl.BlockSpec((gpb * npg, _D), lambda i: (i, 0))],
        out_specs=pl.BlockSpec((gpb, _D), lambda i: (i, 0)),
        out_shape=jax.ShapeDtypeStruct((grid * gpb, _D), jnp.float32),
    )(x)
    return out[:g]


def _sc_gather_rows(table, idx):
    B = idx.shape[0]
    D = table.shape[1]
    bpw = B // _NW
    mesh = plsc.VectorSubcoreMesh(core_axis_name="c", subcore_axis_name="s")

    @functools.partial(
        pl.kernel, mesh=mesh,
        out_type=jax.ShapeDtypeStruct((B, D), jnp.float32),
        compiler_params=pltpu.CompilerParams(needs_layout_passes=False),
        scratch_types=[
            pltpu.VMEM((bpw,), jnp.int32),
            pltpu.VMEM((bpw, D), jnp.float32),
            pltpu.SemaphoreType.DMA,
        ],
    )
    def k(table_hbm, idx_hbm, out_hbm, idx_v, rows_v, sem):
        base = _wid() * bpw
        pltpu.sync_copy(idx_hbm.at[pl.ds(base, bpw)], idx_v)
        pltpu.async_copy(table_hbm.at[idx_v], rows_v, sem).wait()
        pltpu.sync_copy(rows_v, out_hbm.at[pl.ds(base, bpw)])

    return k(table, idx)


def _sc_alpha(q, k, qe2, attr16, srcs, dsts, rowptr_pad, ws, npt, nptp, esz):
    scale = 1.0 / np.sqrt(_D)
    mesh = plsc.VectorSubcoreMesh(core_axis_name="c", subcore_axis_name="s")
    rplen = nptp + 16

    @functools.partial(
        pl.kernel, mesh=mesh,
        out_type=jax.ShapeDtypeStruct((esz,), jnp.float32),
        compiler_params=pltpu.CompilerParams(needs_layout_passes=False),
        scratch_types=[
            pltpu.VMEM((_CH, _D), jnp.float32),
            pltpu.VMEM((_CH, _D), jnp.float32),
            pltpu.VMEM((npt, _AW), jnp.float32),
            pltpu.VMEM((_CH, _AW), jnp.float32),
            pltpu.VMEM((_LBUF,), jnp.float32),
            pltpu.VMEM((nptp,), jnp.float32),
            pltpu.VMEM((nptp,), jnp.float32),
            pltpu.VMEM((_CH,), jnp.float32),
            pltpu.VMEM((_CH,), jnp.int32),
            pltpu.VMEM((_CH,), jnp.int32),
            pltpu.VMEM((rplen,), jnp.int32),
            pltpu.VMEM((16,), jnp.int32),
            pltpu.SemaphoreType.DMA,
        ],
    )
    def ka(q_hbm, k_hbm, qe_hbm, at_hbm, src_hbm, dst_hbm, rp_hbm, ws_hbm,
           al_hbm, qbuf, kbuf, qeloc, abuf, lbuf, m_v, den_v, aout,
           sidx, didx, rp_v, wsrow, sem):
        w = _wid()
        i16 = _iota16()
        pltpu.sync_copy(ws_hbm.at[pl.ds(pl.multiple_of(w * 16, 8), 16)], wsrow)
        wsv = wsrow[...]
        e_lo = _sget(wsv, 0)
        e_hi = _sget(wsv, 1)
        a_lo = pl.multiple_of(_sget(wsv, 2), 8)
        n0 = pl.multiple_of(_sget(wsv, 3), 8)
        rp_base = pl.multiple_of(_sget(wsv, 4), 8)
        shift = w * 128
        rpoff = n0 - rp_base
        cnt = e_hi - a_lo
        nwin = (cnt + _LBUF - 1) // _LBUF

        pltpu.sync_copy(rp_hbm.at[pl.ds(rp_base, rplen)], rp_v)
        pltpu.sync_copy(qe_hbm.at[pl.ds(n0, npt)], qeloc)

        def logits_chunk(ch, wbase):
            ebase = pl.multiple_of(wbase + ch * _CH, 8)
            pltpu.sync_copy(src_hbm.at[pl.ds(ebase, _CH)], sidx)
            pltpu.sync_copy(dst_hbm.at[pl.ds(ebase, _CH)], didx)
            pltpu.sync_copy(at_hbm.at[pl.ds(ebase, _CH)], abuf)
            pltpu.async_copy(k_hbm.at[sidx], kbuf, sem).wait()
            pltpu.async_copy(q_hbm.at[didx], qbuf, sem).wait()
            for g in range(_CH // 16):
                rows = g * 16 + i16
                dloc = jnp.clip(plsc.load_gather(didx, [rows]) - n0, 0, npt - 1)
                def dblk(j, acc):
                    for dd in range(16):
                        col = jnp.full((16,), j * 16 + dd, jnp.int32)
                        qv = plsc.load_gather(qbuf, [rows, col])
                        kv = plsc.load_gather(kbuf, [rows, col])
                        acc = acc + qv * kv
                    return acc
                acc = lax.fori_loop(0, _D // 16, dblk, jnp.zeros((16,), jnp.float32))
                for c in range(5):
                    colc = jnp.full((16,), c, jnp.int32)
                    acc = acc + (plsc.load_gather(qeloc, [dloc, colc])
                                 * plsc.load_gather(abuf, [rows, colc]))
                plsc.store_scatter(lbuf, [ch * _CH + rows], acc * scale)

        def initg(grp, _):
            nodes = grp * 16 + i16
            plsc.store_scatter(m_v, [nodes], jnp.full((16,), -3e38, jnp.float32))
            plsc.store_scatter(den_v, [nodes], jnp.zeros((16,), jnp.float32))
            return 0
        lax.fori_loop(0, nptp // 16, initg, 0)

        def win_body(win, _):
            wbase = a_lo + win * _LBUF
            wlen = jnp.minimum(_LBUF, cnt - win * _LBUF)
            nch = (wlen + _CH - 1) // _CH

            def ch_body(ch, _):
                logits_chunk(ch, wbase)
                return 0
            lax.fori_loop(0, nch, ch_body, 0)

            def grp_body(grp, _):
                nodes = grp * 16 + i16
                rpi = nodes + rpoff
                s_e = plsc.load_gather(rp_v, [rpi])
                e_e = plsc.load_gather(rp_v, [rpi + 1])
                lo = jnp.clip(s_e - wbase, 0, wlen)
                hi = jnp.clip(e_e - wbase, 0, wlen)
                deg = hi - lo
                maxdeg = _smax(deg)
                m0 = plsc.load_gather(m_v, [nodes])
                d0 = plsc.load_gather(den_v, [nodes])

                def jbody(j, md):
                    m, den = md
                    idx = jnp.clip(lo + j, 0, _LBUF - 1)
                    lv = plsc.load_gather(lbuf, [idx])
                    msk = j < deg
                    lvm = jnp.where(msk, lv, -3e38)
                    m2 = jnp.maximum(m, lvm)
                    den2 = den * jnp.exp(m - m2) + jnp.where(
                        msk, jnp.exp(lvm - m2), 0.0)
                    return (m2, den2)

                m1, d1 = lax.fori_loop(0, maxdeg, jbody, (m0, d0))
                plsc.store_scatter(m_v, [nodes], m1)
                plsc.store_scatter(den_v, [nodes], d1)
                return 0
            lax.fori_loop(0, nptp // 16, grp_body, 0)
            return 0
        lax.fori_loop(0, nwin, win_body, 0)

        def rg(grp, _):
            nodes = grp * 16 + i16
            d = plsc.load_gather(den_v, [nodes])
            plsc.store_scatter(den_v, [nodes], 1.0 / (d + 1e-16))
            return 0
        lax.fori_loop(0, nptp // 16, rg, 0)

        def awin_body(win, _):
            wbase = a_lo + win * _LBUF
            wlen = jnp.minimum(_LBUF, cnt - win * _LBUF)
            nch = (wlen + _CH - 1) // _CH

            def ch_body(ch, _):
                @pl.when(nwin > 1)
                def _():
                    logits_chunk(ch, wbase)
                pltpu.sync_copy(dst_hbm.at[pl.ds(
                    pl.multiple_of(wbase + ch * _CH, 8), _CH)], didx)
                for g in range(_CH // 16):
                    rows = g * 16 + i16
                    lv = plsc.load_gather(lbuf, [ch * _CH + rows])
                    dl = jnp.clip(plsc.load_gather(didx, [rows]) - n0, 0, nptp - 1)
                    mv = plsc.load_gather(m_v, [dl])
                    rd = plsc.load_gather(den_v, [dl])
                    plsc.store_scatter(aout, [rows], jnp.exp(lv - mv) * rd)
                pltpu.sync_copy(aout, al_hbm.at[pl.ds(
                    pl.multiple_of(wbase + ch * _CH + shift, 8), _CH)])
                return 0
            lax.fori_loop(0, nch, ch_body, 0)
            return 0
        lax.fori_loop(0, nwin, awin_body, 0)

    return ka(q, k, qe2, attr16, srcs, dsts, rowptr_pad, ws)


def _sc_aggregate(v, attr16, alpha, srcs, dsts, ws, zeros64, npt, nptp):
    mesh = plsc.VectorSubcoreMesh(core_axis_name="c", subcore_axis_name="s")
    hpt = npt // 2

    @functools.partial(
        pl.kernel, mesh=mesh,
        out_type=[
            jax.ShapeDtypeStruct((_NW * npt, _D), jnp.float32),
            jax.ShapeDtypeStruct((_NW * npt, _AW), jnp.float32),
        ],
        compiler_params=pltpu.CompilerParams(needs_layout_passes=False),
        scratch_types=[
            pltpu.VMEM((hpt + 8, _ROW), jnp.float32),
            pltpu.VMEM((_CH, _D), jnp.float32),
            pltpu.VMEM((_CH, _AW), jnp.float32),
            pltpu.VMEM((_CH,), jnp.float32),
            pltpu.VMEM((_CH,), jnp.int32),
            pltpu.VMEM((_CH,), jnp.int32),
            pltpu.VMEM((16,), jnp.int32),
            pltpu.SemaphoreType.DMA,
        ],
    )
    def kb(v_hbm, at_hbm, al_hbm, src_hbm, dst_hbm, ws_hbm, z_hbm,
           aggv_hbm, awa_hbm, acc, vbuf, abuf, av, sidx, didx, wsrow, sem):
        w = _wid()
        i16 = _iota16()
        pltpu.sync_copy(ws_hbm.at[pl.ds(pl.multiple_of(w * 16, 8), 16)], wsrow)
        wsv = wsrow[...]
        shift = w * 128

        for h in range(2):
            e_lo = _sget(wsv, 5 + h)
            e_hi = _sget(wsv, 6 + h)
            a_lo = pl.multiple_of((e_lo // 8) * 8, 8)
            n0 = pl.multiple_of(_sget(wsv, 3) + h * hpt, 8)
            cnt = e_hi - a_lo
            nch = (cnt + _CH - 1) // _CH

            nz = (hpt + 63) // 64
            for zi in range(nz):
                off = min(zi * 64, hpt - 64)
                pltpu.sync_copy(z_hbm, acc.at[pl.ds(off, 64)])
            pltpu.sync_copy(z_hbm.at[pl.ds(0, 8)], acc.at[pl.ds(hpt, 8)])

            def ch_body(ch, _):
                ebase = pl.multiple_of(a_lo + ch * _CH, 8)
                pltpu.sync_copy(src_hbm.at[pl.ds(ebase, _CH)], sidx)
                pltpu.sync_copy(dst_hbm.at[pl.ds(ebase, _CH)], didx)
                pltpu.sync_copy(at_hbm.at[pl.ds(ebase, _CH)], abuf)
                pltpu.sync_copy(al_hbm.at[pl.ds(
                    pl.multiple_of(ebase + shift, 8), _CH)], av)
                pltpu.async_copy(v_hbm.at[sidx], vbuf, sem).wait()

                def e_body(e, _):
                    ev = jnp.full((16,), e, jnp.int32)
                    eg = ebase + e
                    valid = (eg >= e_lo) & (eg < e_hi)
                    dv = plsc.load_gather(didx, [ev])
                    row = jnp.where(valid, jnp.clip(dv - n0, 0, hpt - 1), hpt)
                    asp = plsc.load_gather(av, [ev])
                    for j in range(_D // 16):
                        col = j * 16 + i16
                        vv = plsc.load_gather(vbuf, [ev, col])
                        plsc.addupdate_scatter(acc, [row, col], vv * asp)
                    at = plsc.load_gather(abuf, [ev, i16])
                    plsc.addupdate_scatter(acc, [row, _D + i16], at * asp)
                    return 0
                lax.fori_loop(0, _CH, e_body, 0)
                return 0
            lax.fori_loop(0, nch, ch_body, 0)

            pltpu.sync_copy(acc.at[pl.ds(0, hpt), pl.ds(0, _D)],
                            aggv_hbm.at[pl.ds(n0, hpt)])
            pltpu.sync_copy(acc.at[pl.ds(0, hpt), pl.ds(_D, _AW)],
                            awa_hbm.at[pl.ds(n0, hpt)])

    return kb(v, attr16, alpha, srcs, dsts, ws, zeros64)


def kernel(x, edge_attr, W0, b0, Wq, bq, Wk, bk, Wv, bv, We, be, Ws, bs,
           Ws1, bs1, Ws2, bs2, Ws3, bs3, Wm1, bm1, Wm2, bm2, Wm3, bm3,
           edge_index, batch, stems, stems_batch, slices_x):
    n = x.shape[0]
    E = edge_index.shape[1]
    G = slices_x.shape[0] - 1
    npg = n // G
    npt = -(-(-(-n // _NW)) // 8) * 8
    nptp = -(-npt // 16) * 16
    scale_pad = _NW * npt

    src = edge_index[0]
    dst = edge_index[1]

    perm = jnp.argsort(dst)
    dsts = dst[perm]
    srcs = src[perm]
    rowptr = jnp.searchsorted(dsts, jnp.arange(n + 1, dtype=jnp.int32)
                              ).astype(jnp.int32)
    epad = 256
    srcs_p = jnp.pad(srcs, (0, epad))
    dsts_p = jnp.pad(dsts, (0, epad))
    attr16 = jnp.zeros((E + epad, _AW), jnp.float32)
    attr16 = attr16.at[:E, :4].set(edge_attr[perm])
    attr16 = attr16.at[:E, 4].set(1.0)
    rp_pad_len = ((_NW - 1) * npt // 8) * 8 + nptp + 16 + 8
    rowptr_pad = jnp.full((rp_pad_len,), E, jnp.int32)
    rowptr_pad = rowptr_pad.at[:n + 1].set(rowptr)

    n0s = jnp.arange(_NW, dtype=jnp.int32) * npt
    nends = jnp.minimum(n0s + npt, n)
    e_lo = rowptr[jnp.minimum(n0s, n)]
    e_hi = rowptr[nends]
    a_lo = (e_lo // 8) * 8
    rp_base = (n0s // 8) * 8
    ws_tab = jnp.zeros((_NW, 16), jnp.int32)
    ws_tab = ws_tab.at[:, 0].set(e_lo).at[:, 1].set(e_hi)
    ws_tab = ws_tab.at[:, 2].set(a_lo).at[:, 3].set(n0s).at[:, 4].set(rp_base)
    hpt = npt // 2
    for h in range(3):
        ws_tab = ws_tab.at[:, 5 + h].set(
            rowptr[jnp.minimum(n0s + h * hpt, n)])
    ws_tab = ws_tab.reshape(-1)

    esz = E + 128 * _NW + 256
    zeros64 = jnp.zeros((64, _ROW), jnp.float32)

    WeT16 = jnp.zeros((_D, _AW), jnp.float32)
    WeT16 = WeT16.at[:, :4].set(We.T).at[:, 4].set(be)
    Wfold = jnp.zeros((_AW, _D), jnp.float32)
    Wfold = Wfold.at[:4, :].set(We).at[4, :].set(be)
    W3 = jnp.concatenate([Wq, Wk, Wv], axis=1)
    b3 = jnp.concatenate([bq, bk, bv])

    out = _mm(x, W0, b0, act=True)
    for _ in range(12):
        q, k, v, qe2 = _qkv(out, W3, b3, WeT16, n_out=scale_pad)
        alpha = _sc_alpha(q, k, qe2, attr16, srcs_p, dsts_p, rowptr_pad,
                          ws_tab, npt, nptp, esz)
        aggv, awa = _sc_aggregate(v, attr16, alpha, srcs_p, dsts_p, ws_tab,
                                  zeros64, npt, nptp)
        out = _update(aggv[:n], awa[:n], out, Ws, Wfold, bs)

    global_out = _pool(out, G, npg)
    stem_idx = slices_x[stems_batch] + stems
    S = stem_idx.shape[0]
    Sp = ((S + 255) // 256) * 256
    stem_atom_out = _sc_gather_rows(out, jnp.pad(stem_idx, (0, Sp - S)))[:S]
    glob_rows = _sc_gather_rows(global_out,
                                jnp.pad(stems_batch, (0, Sp - S)))[:S]
    stem_in = jnp.concatenate([stem_atom_out, glob_rows], axis=1)
    h = _mm(stem_in, Ws1, bs1, act=True, bm=512)
    h = _mm(h, Ws2, bs2, act=True, bm=512)
    per_stem_out = _mm(h, Ws3, bs3, bm=512)
    g1 = _mm(global_out, Wm1, bm1, act=True, bm=64)
    g1 = _mm(g1, Wm2, bm2, act=True, bm=64)
    per_mol_out = _mm(g1, Wm3, bm3, bm=64)
    return per_stem_out, per_mol_out
